# baseline (speedup 1.0000x reference)
import functools

import jax
import jax.numpy as jnp
from jax import lax
from jax.experimental import pallas as pl
from jax.experimental.pallas import tpu as pltpu
from jax.experimental.pallas import tpu_sc as plsc

F32 = jnp.float32
L = 128
NO = 2000
NC = 10000
NV = 12000
ACC_R = 12800
DUMMY = 12416
NW = 32
EM_P = 163840
EW_P = 40960
M_CH = 40
W_CH = 10
BN = 1200
BE = 2048

@functools.cache
def _sc_mesh():
    return plsc.VectorSubcoreMesh(core_axis_name="c", subcore_axis_name="s")


def _dot(a, b):
    return jnp.dot(a, b, preferred_element_type=F32)


def _bdot(a, b):
    return jnp.dot(a.astype(jnp.bfloat16), b, preferred_element_type=F32)


def _ln(y, g, b):
    mu = jnp.mean(y, axis=-1, keepdims=True)
    var = jnp.mean((y - mu) * (y - mu), axis=-1, keepdims=True)
    return (y - mu) * lax.rsqrt(var + 1e-5) * g + b


def _wfull(i):
    return pl.BlockSpec(None, lambda *a: tuple(0 for _ in range(i)))


def _w2(shape):
    return pl.BlockSpec(shape, lambda i: (0, 0))


def _w3(shape):
    return pl.BlockSpec(shape, lambda i: (0, 0, 0))


def _enc_node_body(xo, xc, w1, b1, w2, b2, w3, b3, g, bln, wp,
                   v_ref, p0, p1, p2, p3):
    i = pl.program_id(0)
    x = jnp.where(i < 2, xo[...], xc[...])
    _enc_proj_body_from(x, w1, b1, w2, b2, w3, b3, g, bln, wp,
                        v_ref, p0, p1, p2, p3)


def _enc_proj_body_from(x, w1, b1, w2, b2, w3, b3, g, bln, wp,
                        v_ref, p0, p1, p2, p3):
    h = jnp.maximum(_dot(x, w1[...]) + b1[...], 0.0)
    h = jnp.maximum(_dot(h, w2[...]) + b2[...], 0.0)
    v = _ln(_dot(h, w3[...]) + b3[...], g[...], bln[...])
    v_ref[...] = v
    p0[...] = _dot(v, wp[0])
    p1[...] = _dot(v, wp[1])
    p2[...] = _dot(v, wp[2])
    p3[...] = _dot(v, wp[3])


def _enc_node(xo, xc, w, wp):
    w1, w2, w3, b1, b2, b3, g, bln = w
    bn = 1000
    outs = [jax.ShapeDtypeStruct((NV, L), F32) for _ in range(5)]
    return pl.pallas_call(
        _enc_node_body,
        grid=(12,),
        in_specs=[pl.BlockSpec((bn, 12), lambda i: (jnp.minimum(i, 1), 0)),
                  pl.BlockSpec((bn, 12), lambda i: (jnp.maximum(i - 2, 0), 0)),
                  _w2((12, L)), _w2((1, L)), _w2((L, L)), _w2((1, L)),
                  _w2((L, L)), _w2((1, L)), _w2((1, L)), _w2((1, L)),
                  _w3((4, L, L))],
        out_specs=[pl.BlockSpec((bn, L), lambda i: (i, 0))] * 5,
        out_shape=outs,
    )(xo, xc, w1, b1, w2, b2, w3, b3, g, bln, wp)


def _enc_edge_body(x_ref, w1, b1, w2, b2, w3, b3, g, bln, e_ref):
    h = jnp.maximum(_bdot(x_ref[...], w1[...]) + b1[...], 0.0)
    h = jnp.maximum(_bdot(h, w2[...]) + b2[...], 0.0)
    e_ref[...] = _ln(_bdot(h, w3[...]) + b3[...], g[...], bln[...])


def _enc_edge(x, w, n_out):
    w1, w2, w3, b1, b2, b3, g, bln = w
    f = x.shape[1]
    grid = (x.shape[0] + BE - 1) // BE
    return pl.pallas_call(
        _enc_edge_body,
        grid=(grid,),
        in_specs=[pl.BlockSpec((BE, f), lambda i: (i, 0)),
                  _w2((f, L)), _w2((1, L)), _w2((L, L)), _w2((1, L)),
                  _w2((L, L)), _w2((1, L)), _w2((1, L)), _w2((1, L))],
        out_specs=pl.BlockSpec((BE, L), lambda i: (i, 0)),
        out_shape=jax.ShapeDtypeStruct((n_out, L), F32),
    )(x, w1, b1, w2, b2, w3, b3, g, bln)


def _enc_world_body(xd, xi, w1, b1, w2, b2, w3, b3, g, bln, e_ref):
    i = pl.program_id(0)
    x = jnp.where(i < 10, xd[...], xi[...])
    h = jnp.maximum(_bdot(x, w1[...]) + b1[...], 0.0)
    h = jnp.maximum(_bdot(h, w2[...]) + b2[...], 0.0)
    e_ref[...] = _ln(_bdot(h, w3[...]) + b3[...], g[...], bln[...])


def _enc_world(xd, xi, w):
    w1, w2, w3, b1, b2, b3, g, bln = w
    be = 2000
    return pl.pallas_call(
        _enc_world_body,
        grid=(20,),
        in_specs=[pl.BlockSpec((be, 4), lambda i: (jnp.minimum(i, 9), 0)),
                  pl.BlockSpec((be, 4), lambda i: (jnp.maximum(i - 10, 0), 0)),
                  _w2((4, L)), _w2((1, L)), _w2((L, L)), _w2((1, L)),
                  _w2((L, L)), _w2((1, L)), _w2((1, L)), _w2((1, L))],
        out_specs=pl.BlockSpec((be, L), lambda i: (i, 0)),
        out_shape=jax.ShapeDtypeStruct((EW_P, L), F32),
    )(xd, xi, w1, b1, w2, b2, w3, b3, g, bln)


def _edge_upd_body(ga, gb, e_ref, w1c, b1, w2, b2, w3, b3, g, bln,
                   u_ref, en_ref):
    e = e_ref[...]
    x = jnp.maximum(ga[...] + gb[...] + _bdot(e, w1c[...]) + b1[...], 0.0)
    h = jnp.maximum(_bdot(x, w2[...]) + b2[...], 0.0)
    u = _ln(_bdot(h, w3[...]) + b3[...], g[...], bln[...])
    u_ref[...] = u
    en_ref[...] = e + u


def _edge_upd(ga, gb, e, w):
    w1a, w1b, w1c, w2, w3, b1, b2, b3, g, bln = w
    bf = jnp.bfloat16
    w1c, w2, w3 = w1c.astype(bf), w2.astype(bf), w3.astype(bf)
    n = e.shape[0]
    outs = [jax.ShapeDtypeStruct((n, L), F32)] * 2
    blk = pl.BlockSpec((BE, L), lambda i: (i, 0))
    return pl.pallas_call(
        _edge_upd_body,
        grid=(n // BE,),
        in_specs=[blk, blk, blk,
                  _w2((L, L)), _w2((1, L)), _w2((L, L)), _w2((1, L)),
                  _w2((L, L)), _w2((1, L)), _w2((1, L)), _w2((1, L))],
        out_specs=[blk, blk],
        out_shape=outs,
    )(ga, gb, e, w1c, b1, w2, b2, w3, b3, g, bln)


def _node_core(a1, a2, v_ref, w1a, w1b, w1c, b1, w2, b2, w3, b3, g, bln):
    v = v_ref[...]
    A1 = a1[0] + a1[1]
    A2 = a2[0] + a2[1]
    x = jnp.maximum(_dot(A1, w1a[...]) + _dot(A2, w1b[...])
                    + _dot(v, w1c[...]) + b1[...], 0.0)
    h = jnp.maximum(_dot(x, w2[...]) + b2[...], 0.0)
    return v + _ln(_dot(h, w3[...]) + b3[...], g[...], bln[...])


def _node_proj_body(a1, a2, v_ref, w1a, w1b, w1c, b1, w2, b2, w3, b3, g, bln,
                    wp, v_out, p0, p1, p2, p3):
    vn = _node_core(a1, a2, v_ref, w1a, w1b, w1c, b1, w2, b2, w3, b3, g, bln)
    v_out[...] = vn
    p0[...] = _dot(vn, wp[0])
    p1[...] = _dot(vn, wp[1])
    p2[...] = _dot(vn, wp[2])
    p3[...] = _dot(vn, wp[3])


def _node_dec_body(a1, a2, v_ref, w1a, w1b, w1c, b1, w2, b2, w3, b3, g, bln,
                   dw1, db1, dw2, db2, dw3, db3, dec_ref):
    vn = _node_core(a1, a2, v_ref, w1a, w1b, w1c, b1, w2, b2, w3, b3, g, bln)
    d = jnp.maximum(_dot(vn, dw1[...]) + db1[...], 0.0)
    d = jnp.maximum(_dot(d, dw2[...]) + db2[...], 0.0)
    dec_ref[...] = _dot(d, dw3[...]) + db3[...]


_AGG_SPEC = pl.BlockSpec((2, BN, L), lambda i: (0, i, 0))
_NODE_W_SPECS = [_w2((L, L)), _w2((L, L)), _w2((L, L)), _w2((1, L)),
                 _w2((L, L)), _w2((1, L)), _w2((L, L)), _w2((1, L)),
                 _w2((1, L)), _w2((1, L))]


def _node_proj(agg1, agg2, v, w, wp):
    w1a, w1b, w1c, w2, w3, b1, b2, b3, g, bln = w
    blk = pl.BlockSpec((BN, L), lambda i: (i, 0))
    outs = [jax.ShapeDtypeStruct((NV, L), F32)] * 5
    return pl.pallas_call(
        _node_proj_body,
        grid=(NV // BN,),
        in_specs=[_AGG_SPEC, _AGG_SPEC, blk] + _NODE_W_SPECS + [_w3((4, L, L))],
        out_specs=[blk] * 5,
        out_shape=outs,
    )(agg1, agg2, v, w1a, w1b, w1c, b1, w2, b2, w3, b3, g, bln, wp)


def _node_dec(agg1, agg2, v, w, dw):
    w1a, w1b, w1c, w2, w3, b1, b2, b3, g, bln = w
    dw1, dw2, dw3, db1, db2, db3 = dw
    blk = pl.BlockSpec((BN, L), lambda i: (i, 0))
    return pl.pallas_call(
        _node_dec_body,
        grid=(NV // BN,),
        in_specs=[_AGG_SPEC, _AGG_SPEC, blk] + _NODE_W_SPECS
        + [_w2((L, L)), _w2((1, L)), _w2((L, L)), _w2((1, L)),
           _w2((L, 8)), _w2((1, 8))],
        out_specs=pl.BlockSpec((BN, 8), lambda i: (i, 0)),
        out_shape=jax.ShapeDtypeStruct((NV, 8), F32),
    )(agg1, agg2, v, w1a, w1b, w1c, b1, w2, b2, w3, b3, g, bln,
      dw1, db1, dw2, db2, dw3, db3)


GCH = 64


def _pipe_gather(tbl, ivm, out, nch, base, bufs, gsems, ssems):
    def idx(c):
        return ivm.at[c // 2, pl.ds((c % 2) * 64, GCH)]

    for k in range(2):
        pltpu.async_copy(tbl.at[idx(k)], bufs[k], gsems[k])

    def body(i, carry):
        c0 = i * 2
        for k in range(2):
            c = c0 + k
            pltpu.make_async_copy(tbl.at[idx(c)], bufs[k], gsems[k]).wait()
            dst = out.at[pl.ds(base + c * GCH, GCH)]
            pltpu.async_copy(bufs[k], dst, ssems[k])

            @pl.when(c + 2 < nch)
            def _():
                pltpu.make_async_copy(bufs[k], dst, ssems[k]).wait()
                pltpu.async_copy(tbl.at[idx(c + 2)], bufs[k], gsems[k])
        return carry
    lax.fori_loop(0, nch // 2, body, 0)
    for k in range(2):
        c = nch - 2 + k
        pltpu.make_async_copy(
            bufs[k], out.at[pl.ds(base + c * GCH, GCH)], ssems[k]).wait()


def _stage(src, dst, s):
    @pl.when(s < 15)
    def _():
        pltpu.sync_copy(src.at[pl.ds(s * 752, 752)],
                        dst.at[pl.ds(s * 752, 752)])

    @pl.when(s == 15)
    def _():
        pltpu.sync_copy(src.at[pl.ds(11280, 720)],
                        dst.at[pl.ds(11280, 720)])


def _gather_body(p0, p1, p2, p3, imA, imB, iwA, iwB,
                 gmA, gmB, gwA, gwB,
                 vimA, vimB, viwA, viwB,
                 b0, b1, tbl_sh, g0, g1, s0, s1):
    c = lax.axis_index("c")
    s = lax.axis_index("s")
    wid = c * 16 + s
    pltpu.sync_copy(imA.at[wid], vimA)
    pltpu.sync_copy(imB.at[wid], vimB)
    pltpu.sync_copy(iwA.at[wid], viwA)
    pltpu.sync_copy(iwB.at[wid], viwB)
    bufs = (b0, b1)
    gsems = (g0, g1)
    ssems = (s0, s1)
    mbase = wid * (M_CH * 128)
    wbase = wid * (W_CH * 128)
    mch = (M_CH * 128) // GCH
    wch = (W_CH * 128) // GCH
    for tbl, ivm, out, nch, base in ((p0, vimA, gmA, mch, mbase),
                                     (p1, vimB, gmB, mch, mbase),
                                     (p2, viwA, gwA, wch, wbase),
                                     (p3, viwB, gwB, wch, wbase)):
        _stage(tbl, tbl_sh, s)
        plsc.subcore_barrier()
        _pipe_gather(tbl_sh, ivm, out, nch, base, bufs, gsems, ssems)
        plsc.subcore_barrier()


@functools.cache
def _sc_gather_kernel():
    return pl.kernel(
        _gather_body,
        out_type=[jax.ShapeDtypeStruct((EM_P, L), F32),
                  jax.ShapeDtypeStruct((EM_P, L), F32),
                  jax.ShapeDtypeStruct((EW_P, L), F32),
                  jax.ShapeDtypeStruct((EW_P, L), F32)],
        mesh=_sc_mesh(),
        scratch_types=[pltpu.VMEM((M_CH, 128), jnp.int32),
                       pltpu.VMEM((M_CH, 128), jnp.int32),
                       pltpu.VMEM((W_CH, 128), jnp.int32),
                       pltpu.VMEM((W_CH, 128), jnp.int32)]
        + [pltpu.VMEM((GCH, L), F32)] * 2
        + [pltpu.VMEM_SHARED((NV, L), F32)]
        + [pltpu.SemaphoreType.DMA] * 4,
    )


def _sc_gather(*args):
    return _sc_gather_kernel()(*args)


def _make_scatter(e_p, nch):
    npt = e_p // NW
    sch = 64

    def body(u_ref, idx_ref, out_ref, ivm, d0, d1, acc, rs0, rs1, ws0, ws1):
        c = lax.axis_index("c")
        s = lax.axis_index("s")
        wid = c * 16 + s
        bufs = (d0, d1)

        def zb(i, carry):
            r = i // 8
            col = (i % 8) * 16
            d0[r, pl.ds(col, 16)] = jnp.zeros((16,), F32)
            return carry
        lax.fori_loop(0, 512, zb, 0)

        def za(r, carry):
            pltpu.sync_copy(d0, acc.at[pl.ds(s * 800 + r * sch, sch)])
            return carry
        lax.fori_loop(0, 12, za, 0)
        pltpu.sync_copy(d0.at[pl.ds(0, 32)],
                        acc.at[pl.ds(s * 800 + 768, 32)])
        plsc.subcore_barrier()

        pltpu.sync_copy(idx_ref.at[wid], ivm)

        def src(ci):
            return u_ref.at[pl.ds(wid * npt + ci * sch, sch)]

        rsems = (rs0, rs1)
        wsems = (ws0, ws1)
        for k in range(2):
            pltpu.async_copy(src(k), bufs[k], rsems[k])

        def sc(i, carry):
            c0 = i * 2
            for k in range(2):
                ci = c0 + k
                pltpu.make_async_copy(src(ci), bufs[k], rsems[k]).wait()
                pltpu.async_copy(bufs[k], acc.at[ivm.at[ci]], wsems[k],
                                 add=True)

                @pl.when(ci + 2 < nch)
                def _():
                    pltpu.make_async_copy(bufs[k], acc.at[ivm.at[ci]],
                                          wsems[k]).wait()
                    pltpu.async_copy(src(ci + 2), bufs[k], rsems[k])
            return carry
        lax.fori_loop(0, nch // 2, sc, 0)
        for k in range(2):
            ci = nch - 2 + k
            pltpu.make_async_copy(bufs[k], acc.at[ivm.at[ci]],
                                  wsems[k]).wait()
        plsc.subcore_barrier()

        pltpu.sync_copy(acc.at[pl.ds(s * 800, 800)],
                        out_ref.at[pl.ds(c * ACC_R + s * 800, 800)])

    @functools.cache
    def build():
        return pl.kernel(
            body,
            out_type=jax.ShapeDtypeStruct((2 * ACC_R, L), F32),
            mesh=_sc_mesh(),
            scratch_types=[pltpu.VMEM((nch, sch), jnp.int32),
                           pltpu.VMEM((sch, L), F32),
                           pltpu.VMEM((sch, L), F32),
                           pltpu.VMEM_SHARED((ACC_R, L), F32)]
            + [pltpu.SemaphoreType.DMA] * 4,
        )

    def call(u, idx3):
        return build()(u, idx3)

    return call


_sc_scatter_mesh = _make_scatter(EM_P, 80)
_sc_scatter_world = _make_scatter(EW_P, 20)


def _mw(mp):
    W1, W2, W3 = mp['Ws']
    b1, b2, b3 = [b.reshape(1, L) for b in mp['bs']]
    return W1, W2, W3, b1, b2, b3, mp['g'].reshape(1, L), mp['b2'].reshape(1, L)


def _enc_w(mp, in_pad):
    W1, W2, W3, b1, b2, b3, g, bln = _mw(mp)
    W1 = jnp.pad(W1, ((0, in_pad - W1.shape[0]), (0, 0)))
    return W1, W2, W3, b1, b2, b3, g, bln


def _enc_w_bf(mp):
    W1, W2, W3, b1, b2, b3, g, bln = _mw(mp)
    bf = jnp.bfloat16
    return W1.astype(bf), W2.astype(bf), W3.astype(bf), b1, b2, b3, g, bln


def _blk_w(mp):
    W1, W2, W3, b1, b2, b3, g, bln = _mw(mp)
    return W1[:L], W1[L:2 * L], W1[2 * L:], W2, W3, b1, b2, b3, g, bln


def _proj_stack(blk):
    wm = blk['mesh_edge']['Ws'][0]
    ww = blk['world_edge']['Ws'][0]
    return jnp.stack([wm[:L], wm[L:2 * L], ww[:L], ww[L:2 * L]])


def kernel(cloth_features, obstacle_features, mesh_edge_features,
           world_direct_features, world_inverse_features, params,
           mesh_edge_index, world_direct_edge_index, world_inverse_edge_index):
    p = params


    mdst = mesh_edge_index[1] + NO
    msrc = mesh_edge_index[0] + NO
    im_a = jnp.pad(mdst, (0, EM_P - 160000)).reshape(NW, M_CH, 128)
    im_b = jnp.pad(msrc, (0, EM_P - 160000)).reshape(NW, M_CH, 128)
    im_s = jnp.pad(mdst, (0, EM_P - 160000),
                   constant_values=DUMMY).reshape(NW, 80, 64)
    wdst = jnp.concatenate([world_direct_edge_index[1],
                            world_inverse_edge_index[1] + NO])
    wsrc = jnp.concatenate([world_direct_edge_index[0] + NO,
                            world_inverse_edge_index[0]])
    iw_a = jnp.pad(wdst, (0, EW_P - 40000)).reshape(NW, W_CH, 128)
    iw_b = jnp.pad(wsrc, (0, EW_P - 40000)).reshape(NW, W_CH, 128)
    iw_s = jnp.pad(wdst, (0, EW_P - 40000),
                   constant_values=DUMMY).reshape(NW, 20, 64)

    enc_node_w = _mw(p['node_encoder'])
    enc_mesh_w = _enc_w_bf(p['mesh_encoder'])
    enc_world_w = _enc_w_bf(p['world_encoder'])
    blk_w = [{'mesh': _blk_w(b['mesh_edge']),
              'world': _blk_w(b['world_edge']),
              'node': _blk_w(b['node'])} for b in p['blocks']]
    dW1, dW2, dW3 = p['decoder']['Ws']
    dec_w = (dW1, dW2, jnp.pad(dW3, ((0, 0), (0, 8 - dW3.shape[1]))),
             p['decoder']['bs'][0].reshape(1, L),
             p['decoder']['bs'][1].reshape(1, L),
             jnp.pad(p['decoder']['bs'][2].reshape(1, 3), ((0, 0), (0, 5))))

    v, p0, p1, p2, p3 = _enc_node(obstacle_features, cloth_features,
                                  enc_node_w, _proj_stack(p['blocks'][0]))
    em = _enc_edge(mesh_edge_features, enc_mesh_w, EM_P)
    ew = _enc_world(world_direct_features, world_inverse_features,
                    enc_world_w)

    for s in range(len(p['blocks'])):
        gmA, gmB, gwA, gwB = _sc_gather(p0, p1, p2, p3, im_a, im_b, iw_a, iw_b)
        em_upd, em = _edge_upd(gmA, gmB, em, blk_w[s]['mesh'])
        ew_upd, ew = _edge_upd(gwA, gwB, ew, blk_w[s]['world'])
        agg1 = _sc_scatter_mesh(em_upd, im_s).reshape(2, ACC_R, L)
        agg2 = _sc_scatter_world(ew_upd, iw_s).reshape(2, ACC_R, L)
        if s + 1 < len(p['blocks']):
            v, p0, p1, p2, p3 = _node_proj(agg1, agg2, v, blk_w[s]['node'],
                                           _proj_stack(p['blocks'][s + 1]))
        else:
            dec = _node_dec(agg1, agg2, v, blk_w[s]['node'], dec_w)

    return dec[NO:NV, :3]


# speedup vs baseline: 1.0496x; 1.0496x over previous
import functools

import jax
import jax.numpy as jnp
from jax import lax
from jax.experimental import pallas as pl
from jax.experimental.pallas import tpu as pltpu
from jax.experimental.pallas import tpu_sc as plsc

F32 = jnp.float32
L = 128
NO = 2000
NC = 10000
NV = 12000
ACC_R = 12800
DUMMY = 12416
NW = 32
EM_P = 163840
EW_P = 40960
M_CH = 40
W_CH = 10
BN = 1200
BE = 2048

@functools.cache
def _sc_mesh():
    return plsc.VectorSubcoreMesh(core_axis_name="c", subcore_axis_name="s")


def _dot(a, b):
    return jnp.dot(a, b, preferred_element_type=F32)


def _bdot(a, b):
    return jnp.dot(a.astype(jnp.bfloat16), b, preferred_element_type=F32)


def _ln(y, g, b):
    mu = jnp.mean(y, axis=-1, keepdims=True)
    var = jnp.mean((y - mu) * (y - mu), axis=-1, keepdims=True)
    return (y - mu) * lax.rsqrt(var + 1e-5) * g + b


def _wfull(i):
    return pl.BlockSpec(None, lambda *a: tuple(0 for _ in range(i)))


def _w2(shape):
    return pl.BlockSpec(shape, lambda i: (0, 0))


def _w3(shape):
    return pl.BlockSpec(shape, lambda i: (0, 0, 0))


def _enc_node_body(xo, xc, w1, b1, w2, b2, w3, b3, g, bln, wp,
                   v_ref, p0, p1, p2, p3):
    i = pl.program_id(0)
    x = jnp.where(i < 2, xo[...], xc[...])
    _enc_proj_body_from(x, w1, b1, w2, b2, w3, b3, g, bln, wp,
                        v_ref, p0, p1, p2, p3)


def _enc_proj_body_from(x, w1, b1, w2, b2, w3, b3, g, bln, wp,
                        v_ref, p0, p1, p2, p3):
    h = jnp.maximum(_dot(x, w1[...]) + b1[...], 0.0)
    h = jnp.maximum(_dot(h, w2[...]) + b2[...], 0.0)
    v = _ln(_dot(h, w3[...]) + b3[...], g[...], bln[...])
    v_ref[...] = v
    p0[...] = _dot(v, wp[0])
    p1[...] = _dot(v, wp[1])
    p2[...] = _dot(v, wp[2])
    p3[...] = _dot(v, wp[3])


def _enc_node(xo, xc, w, wp):
    w1, w2, w3, b1, b2, b3, g, bln = w
    bn = 1000
    outs = [jax.ShapeDtypeStruct((NV, L), F32) for _ in range(5)]
    return pl.pallas_call(
        _enc_node_body,
        grid=(12,),
        in_specs=[pl.BlockSpec((bn, 12), lambda i: (jnp.minimum(i, 1), 0)),
                  pl.BlockSpec((bn, 12), lambda i: (jnp.maximum(i - 2, 0), 0)),
                  _w2((12, L)), _w2((1, L)), _w2((L, L)), _w2((1, L)),
                  _w2((L, L)), _w2((1, L)), _w2((1, L)), _w2((1, L)),
                  _w3((4, L, L))],
        out_specs=[pl.BlockSpec((bn, L), lambda i: (i, 0))] * 5,
        out_shape=outs,
    )(xo, xc, w1, b1, w2, b2, w3, b3, g, bln, wp)


def _enc_edge_body(x_ref, w1, b1, w2, b2, w3, b3, g, bln, e_ref):
    h = jnp.maximum(_bdot(x_ref[...], w1[...]) + b1[...], 0.0)
    h = jnp.maximum(_bdot(h, w2[...]) + b2[...], 0.0)
    e_ref[...] = _ln(_bdot(h, w3[...]) + b3[...], g[...],
                     bln[...]).astype(jnp.bfloat16)


def _enc_edge(x, w, n_out):
    w1, w2, w3, b1, b2, b3, g, bln = w
    f = x.shape[1]
    grid = (x.shape[0] + BE - 1) // BE
    return pl.pallas_call(
        _enc_edge_body,
        grid=(grid,),
        in_specs=[pl.BlockSpec((BE, f), lambda i: (i, 0)),
                  _w2((f, L)), _w2((1, L)), _w2((L, L)), _w2((1, L)),
                  _w2((L, L)), _w2((1, L)), _w2((1, L)), _w2((1, L))],
        out_specs=pl.BlockSpec((BE, L), lambda i: (i, 0)),
        out_shape=jax.ShapeDtypeStruct((n_out, L), jnp.bfloat16),
    )(x, w1, b1, w2, b2, w3, b3, g, bln)


def _enc_world_body(xd, xi, w1, b1, w2, b2, w3, b3, g, bln, e_ref):
    i = pl.program_id(0)
    x = jnp.where(i < 10, xd[...], xi[...])
    h = jnp.maximum(_bdot(x, w1[...]) + b1[...], 0.0)
    h = jnp.maximum(_bdot(h, w2[...]) + b2[...], 0.0)
    e_ref[...] = _ln(_bdot(h, w3[...]) + b3[...], g[...],
                     bln[...]).astype(jnp.bfloat16)


def _enc_world(xd, xi, w):
    w1, w2, w3, b1, b2, b3, g, bln = w
    be = 2000
    return pl.pallas_call(
        _enc_world_body,
        grid=(20,),
        in_specs=[pl.BlockSpec((be, 4), lambda i: (jnp.minimum(i, 9), 0)),
                  pl.BlockSpec((be, 4), lambda i: (jnp.maximum(i - 10, 0), 0)),
                  _w2((4, L)), _w2((1, L)), _w2((L, L)), _w2((1, L)),
                  _w2((L, L)), _w2((1, L)), _w2((1, L)), _w2((1, L))],
        out_specs=pl.BlockSpec((be, L), lambda i: (i, 0)),
        out_shape=jax.ShapeDtypeStruct((EW_P, L), jnp.bfloat16),
    )(xd, xi, w1, b1, w2, b2, w3, b3, g, bln)


def _edge_upd_body(ga, gb, e_ref, w1c, b1, w2, b2, w3, b3, g, bln,
                   u_ref, en_ref):
    e = e_ref[...]
    x = jnp.maximum(ga[...] + gb[...] + jnp.dot(
        e, w1c[...], preferred_element_type=F32) + b1[...], 0.0)
    h = jnp.maximum(_bdot(x, w2[...]) + b2[...], 0.0)
    u = _ln(_bdot(h, w3[...]) + b3[...], g[...], bln[...])
    u_ref[...] = u
    en_ref[...] = (e.astype(F32) + u).astype(jnp.bfloat16)


def _edge_upd(ga, gb, e, w):
    w1a, w1b, w1c, w2, w3, b1, b2, b3, g, bln = w
    bf = jnp.bfloat16
    w1c, w2, w3 = w1c.astype(bf), w2.astype(bf), w3.astype(bf)
    n = e.shape[0]
    outs = [jax.ShapeDtypeStruct((n, L), F32),
            jax.ShapeDtypeStruct((n, L), bf)]
    blk = pl.BlockSpec((BE, L), lambda i: (i, 0))
    return pl.pallas_call(
        _edge_upd_body,
        grid=(n // BE,),
        in_specs=[blk, blk, blk,
                  _w2((L, L)), _w2((1, L)), _w2((L, L)), _w2((1, L)),
                  _w2((L, L)), _w2((1, L)), _w2((1, L)), _w2((1, L))],
        out_specs=[blk, blk],
        out_shape=outs,
    )(ga, gb, e, w1c, b1, w2, b2, w3, b3, g, bln)


def _node_core(a1, a2, v_ref, w1a, w1b, w1c, b1, w2, b2, w3, b3, g, bln):
    v = v_ref[...]
    A1 = a1[0] + a1[1]
    A2 = a2[0] + a2[1]
    x = jnp.maximum(_dot(A1, w1a[...]) + _dot(A2, w1b[...])
                    + _dot(v, w1c[...]) + b1[...], 0.0)
    h = jnp.maximum(_dot(x, w2[...]) + b2[...], 0.0)
    return v + _ln(_dot(h, w3[...]) + b3[...], g[...], bln[...])


def _node_proj_body(a1, a2, v_ref, w1a, w1b, w1c, b1, w2, b2, w3, b3, g, bln,
                    wp, v_out, p0, p1, p2, p3):
    vn = _node_core(a1, a2, v_ref, w1a, w1b, w1c, b1, w2, b2, w3, b3, g, bln)
    v_out[...] = vn
    p0[...] = _dot(vn, wp[0])
    p1[...] = _dot(vn, wp[1])
    p2[...] = _dot(vn, wp[2])
    p3[...] = _dot(vn, wp[3])


def _node_dec_body(a1, a2, v_ref, w1a, w1b, w1c, b1, w2, b2, w3, b3, g, bln,
                   dw1, db1, dw2, db2, dw3, db3, dec_ref):
    vn = _node_core(a1, a2, v_ref, w1a, w1b, w1c, b1, w2, b2, w3, b3, g, bln)
    d = jnp.maximum(_dot(vn, dw1[...]) + db1[...], 0.0)
    d = jnp.maximum(_dot(d, dw2[...]) + db2[...], 0.0)
    dec_ref[...] = _dot(d, dw3[...]) + db3[...]


_AGG_SPEC = pl.BlockSpec((2, BN, L), lambda i: (0, i, 0))
_NODE_W_SPECS = [_w2((L, L)), _w2((L, L)), _w2((L, L)), _w2((1, L)),
                 _w2((L, L)), _w2((1, L)), _w2((L, L)), _w2((1, L)),
                 _w2((1, L)), _w2((1, L))]


def _node_proj(agg1, agg2, v, w, wp):
    w1a, w1b, w1c, w2, w3, b1, b2, b3, g, bln = w
    blk = pl.BlockSpec((BN, L), lambda i: (i, 0))
    outs = [jax.ShapeDtypeStruct((NV, L), F32)] * 5
    return pl.pallas_call(
        _node_proj_body,
        grid=(NV // BN,),
        in_specs=[_AGG_SPEC, _AGG_SPEC, blk] + _NODE_W_SPECS + [_w3((4, L, L))],
        out_specs=[blk] * 5,
        out_shape=outs,
    )(agg1, agg2, v, w1a, w1b, w1c, b1, w2, b2, w3, b3, g, bln, wp)


def _node_dec(agg1, agg2, v, w, dw):
    w1a, w1b, w1c, w2, w3, b1, b2, b3, g, bln = w
    dw1, dw2, dw3, db1, db2, db3 = dw
    blk = pl.BlockSpec((BN, L), lambda i: (i, 0))
    return pl.pallas_call(
        _node_dec_body,
        grid=(NV // BN,),
        in_specs=[_AGG_SPEC, _AGG_SPEC, blk] + _NODE_W_SPECS
        + [_w2((L, L)), _w2((1, L)), _w2((L, L)), _w2((1, L)),
           _w2((L, 8)), _w2((1, 8))],
        out_specs=pl.BlockSpec((BN, 8), lambda i: (i, 0)),
        out_shape=jax.ShapeDtypeStruct((NV, 8), F32),
    )(agg1, agg2, v, w1a, w1b, w1c, b1, w2, b2, w3, b3, g, bln,
      dw1, db1, dw2, db2, dw3, db3)


GCH = 64


def _pipe_gather(tbl, ivm, out, nch, base, bufs, gsems, ssems):
    def idx(c):
        return ivm.at[c // 2, pl.ds((c % 2) * 64, GCH)]

    for k in range(2):
        pltpu.async_copy(tbl.at[idx(k)], bufs[k], gsems[k])

    def body(i, carry):
        c0 = i * 2
        for k in range(2):
            c = c0 + k
            pltpu.make_async_copy(tbl.at[idx(c)], bufs[k], gsems[k]).wait()
            dst = out.at[pl.ds(base + c * GCH, GCH)]
            pltpu.async_copy(bufs[k], dst, ssems[k])

            @pl.when(c + 2 < nch)
            def _():
                pltpu.make_async_copy(bufs[k], dst, ssems[k]).wait()
                pltpu.async_copy(tbl.at[idx(c + 2)], bufs[k], gsems[k])
        return carry
    lax.fori_loop(0, nch // 2, body, 0)
    for k in range(2):
        c = nch - 2 + k
        pltpu.make_async_copy(
            bufs[k], out.at[pl.ds(base + c * GCH, GCH)], ssems[k]).wait()


def _stage(src, dst, s):
    @pl.when(s < 15)
    def _():
        pltpu.sync_copy(src.at[pl.ds(s * 752, 752)],
                        dst.at[pl.ds(s * 752, 752)])

    @pl.when(s == 15)
    def _():
        pltpu.sync_copy(src.at[pl.ds(11280, 720)],
                        dst.at[pl.ds(11280, 720)])


def _gather_body(p0, p1, p2, p3, imA, imB, iwA, iwB,
                 gmA, gmB, gwA, gwB,
                 vimA, vimB, viwA, viwB,
                 b0, b1, tbl_sh, g0, g1, s0, s1):
    c = lax.axis_index("c")
    s = lax.axis_index("s")
    wid = c * 16 + s
    pltpu.sync_copy(imA.at[wid], vimA)
    pltpu.sync_copy(imB.at[wid], vimB)
    pltpu.sync_copy(iwA.at[wid], viwA)
    pltpu.sync_copy(iwB.at[wid], viwB)
    bufs = (b0, b1)
    gsems = (g0, g1)
    ssems = (s0, s1)
    mbase = wid * (M_CH * 128)
    wbase = wid * (W_CH * 128)
    mch = (M_CH * 128) // GCH
    wch = (W_CH * 128) // GCH
    for tbl, ivm, out, nch, base in ((p0, vimA, gmA, mch, mbase),
                                     (p1, vimB, gmB, mch, mbase),
                                     (p2, viwA, gwA, wch, wbase),
                                     (p3, viwB, gwB, wch, wbase)):
        _stage(tbl, tbl_sh, s)
        plsc.subcore_barrier()
        _pipe_gather(tbl_sh, ivm, out, nch, base, bufs, gsems, ssems)
        plsc.subcore_barrier()


@functools.cache
def _sc_gather_kernel():
    return pl.kernel(
        _gather_body,
        out_type=[jax.ShapeDtypeStruct((EM_P, L), F32),
                  jax.ShapeDtypeStruct((EM_P, L), F32),
                  jax.ShapeDtypeStruct((EW_P, L), F32),
                  jax.ShapeDtypeStruct((EW_P, L), F32)],
        mesh=_sc_mesh(),
        scratch_types=[pltpu.VMEM((M_CH, 128), jnp.int32),
                       pltpu.VMEM((M_CH, 128), jnp.int32),
                       pltpu.VMEM((W_CH, 128), jnp.int32),
                       pltpu.VMEM((W_CH, 128), jnp.int32)]
        + [pltpu.VMEM((GCH, L), F32)] * 2
        + [pltpu.VMEM_SHARED((NV, L), F32)]
        + [pltpu.SemaphoreType.DMA] * 4,
    )


def _sc_gather(*args):
    return _sc_gather_kernel()(*args)


def _make_scatter(e_p, nch):
    npt = e_p // NW
    sch = 64

    def body(u_ref, idx_ref, out_ref, ivm, d0, d1, acc, rs0, rs1, ws0, ws1):
        c = lax.axis_index("c")
        s = lax.axis_index("s")
        wid = c * 16 + s
        bufs = (d0, d1)

        def zb(i, carry):
            r = i // 8
            col = (i % 8) * 16
            d0[r, pl.ds(col, 16)] = jnp.zeros((16,), F32)
            return carry
        lax.fori_loop(0, 512, zb, 0)

        def za(r, carry):
            pltpu.sync_copy(d0, acc.at[pl.ds(s * 800 + r * sch, sch)])
            return carry
        lax.fori_loop(0, 12, za, 0)
        pltpu.sync_copy(d0.at[pl.ds(0, 32)],
                        acc.at[pl.ds(s * 800 + 768, 32)])
        plsc.subcore_barrier()

        pltpu.sync_copy(idx_ref.at[wid], ivm)

        def src(ci):
            return u_ref.at[pl.ds(wid * npt + ci * sch, sch)]

        rsems = (rs0, rs1)
        wsems = (ws0, ws1)
        for k in range(2):
            pltpu.async_copy(src(k), bufs[k], rsems[k])

        def sc(i, carry):
            c0 = i * 2
            for k in range(2):
                ci = c0 + k
                pltpu.make_async_copy(src(ci), bufs[k], rsems[k]).wait()
                pltpu.async_copy(bufs[k], acc.at[ivm.at[ci]], wsems[k],
                                 add=True)

                @pl.when(ci + 2 < nch)
                def _():
                    pltpu.make_async_copy(bufs[k], acc.at[ivm.at[ci]],
                                          wsems[k]).wait()
                    pltpu.async_copy(src(ci + 2), bufs[k], rsems[k])
            return carry
        lax.fori_loop(0, nch // 2, sc, 0)
        for k in range(2):
            ci = nch - 2 + k
            pltpu.make_async_copy(bufs[k], acc.at[ivm.at[ci]],
                                  wsems[k]).wait()
        plsc.subcore_barrier()

        pltpu.sync_copy(acc.at[pl.ds(s * 800, 800)],
                        out_ref.at[pl.ds(c * ACC_R + s * 800, 800)])

    @functools.cache
    def build():
        return pl.kernel(
            body,
            out_type=jax.ShapeDtypeStruct((2 * ACC_R, L), F32),
            mesh=_sc_mesh(),
            scratch_types=[pltpu.VMEM((nch, sch), jnp.int32),
                           pltpu.VMEM((sch, L), F32),
                           pltpu.VMEM((sch, L), F32),
                           pltpu.VMEM_SHARED((ACC_R, L), F32)]
            + [pltpu.SemaphoreType.DMA] * 4,
        )

    def call(u, idx3):
        return build()(u, idx3)

    return call


_sc_scatter_mesh = _make_scatter(EM_P, 80)
_sc_scatter_world = _make_scatter(EW_P, 20)


def _mw(mp):
    W1, W2, W3 = mp['Ws']
    b1, b2, b3 = [b.reshape(1, L) for b in mp['bs']]
    return W1, W2, W3, b1, b2, b3, mp['g'].reshape(1, L), mp['b2'].reshape(1, L)


def _enc_w(mp, in_pad):
    W1, W2, W3, b1, b2, b3, g, bln = _mw(mp)
    W1 = jnp.pad(W1, ((0, in_pad - W1.shape[0]), (0, 0)))
    return W1, W2, W3, b1, b2, b3, g, bln


def _enc_w_bf(mp):
    W1, W2, W3, b1, b2, b3, g, bln = _mw(mp)
    bf = jnp.bfloat16
    return W1.astype(bf), W2.astype(bf), W3.astype(bf), b1, b2, b3, g, bln


def _blk_w(mp):
    W1, W2, W3, b1, b2, b3, g, bln = _mw(mp)
    return W1[:L], W1[L:2 * L], W1[2 * L:], W2, W3, b1, b2, b3, g, bln


def _proj_stack(blk):
    wm = blk['mesh_edge']['Ws'][0]
    ww = blk['world_edge']['Ws'][0]
    return jnp.stack([wm[:L], wm[L:2 * L], ww[:L], ww[L:2 * L]])


def kernel(cloth_features, obstacle_features, mesh_edge_features,
           world_direct_features, world_inverse_features, params,
           mesh_edge_index, world_direct_edge_index, world_inverse_edge_index):
    p = params


    mdst = mesh_edge_index[1] + NO
    msrc = mesh_edge_index[0] + NO
    im_a = jnp.pad(mdst, (0, EM_P - 160000)).reshape(NW, M_CH, 128)
    im_b = jnp.pad(msrc, (0, EM_P - 160000)).reshape(NW, M_CH, 128)
    im_s = jnp.pad(mdst, (0, EM_P - 160000),
                   constant_values=DUMMY).reshape(NW, 80, 64)
    wdst = jnp.concatenate([world_direct_edge_index[1],
                            world_inverse_edge_index[1] + NO])
    wsrc = jnp.concatenate([world_direct_edge_index[0] + NO,
                            world_inverse_edge_index[0]])
    iw_a = jnp.pad(wdst, (0, EW_P - 40000)).reshape(NW, W_CH, 128)
    iw_b = jnp.pad(wsrc, (0, EW_P - 40000)).reshape(NW, W_CH, 128)
    iw_s = jnp.pad(wdst, (0, EW_P - 40000),
                   constant_values=DUMMY).reshape(NW, 20, 64)

    enc_node_w = _mw(p['node_encoder'])
    enc_mesh_w = _enc_w_bf(p['mesh_encoder'])
    enc_world_w = _enc_w_bf(p['world_encoder'])
    blk_w = [{'mesh': _blk_w(b['mesh_edge']),
              'world': _blk_w(b['world_edge']),
              'node': _blk_w(b['node'])} for b in p['blocks']]
    dW1, dW2, dW3 = p['decoder']['Ws']
    dec_w = (dW1, dW2, jnp.pad(dW3, ((0, 0), (0, 8 - dW3.shape[1]))),
             p['decoder']['bs'][0].reshape(1, L),
             p['decoder']['bs'][1].reshape(1, L),
             jnp.pad(p['decoder']['bs'][2].reshape(1, 3), ((0, 0), (0, 5))))

    v, p0, p1, p2, p3 = _enc_node(obstacle_features, cloth_features,
                                  enc_node_w, _proj_stack(p['blocks'][0]))
    em = _enc_edge(mesh_edge_features, enc_mesh_w, EM_P)
    ew = _enc_world(world_direct_features, world_inverse_features,
                    enc_world_w)

    for s in range(len(p['blocks'])):
        gmA, gmB, gwA, gwB = _sc_gather(p0, p1, p2, p3, im_a, im_b, iw_a, iw_b)
        em_upd, em = _edge_upd(gmA, gmB, em, blk_w[s]['mesh'])
        ew_upd, ew = _edge_upd(gwA, gwB, ew, blk_w[s]['world'])
        agg1 = _sc_scatter_mesh(em_upd, im_s).reshape(2, ACC_R, L)
        agg2 = _sc_scatter_world(ew_upd, iw_s).reshape(2, ACC_R, L)
        if s + 1 < len(p['blocks']):
            v, p0, p1, p2, p3 = _node_proj(agg1, agg2, v, blk_w[s]['node'],
                                           _proj_stack(p['blocks'][s + 1]))
        else:
            dec = _node_dec(agg1, agg2, v, blk_w[s]['node'], dec_w)

    return dec[NO:NV, :3]


# speedup vs baseline: 1.0499x; 1.0003x over previous
import functools

import jax
import jax.numpy as jnp
from jax import lax
from jax.experimental import pallas as pl
from jax.experimental.pallas import tpu as pltpu
from jax.experimental.pallas import tpu_sc as plsc

F32 = jnp.float32
L = 128
NO = 2000
NC = 10000
NV = 12000
ACC_R = 12800
DUMMY = 12416
NW = 32
EM_P = 163840
EW_P = 40960
M_CH = 40
W_CH = 10
BN = 1200
BE = 2048

@functools.cache
def _sc_mesh():
    return plsc.VectorSubcoreMesh(core_axis_name="c", subcore_axis_name="s")


def _dot(a, b):
    return jnp.dot(a, b, preferred_element_type=F32)


def _bdot(a, b):
    return jnp.dot(a.astype(jnp.bfloat16), b, preferred_element_type=F32)


def _ln(y, g, b):
    mu = jnp.mean(y, axis=-1, keepdims=True)
    var = jnp.mean((y - mu) * (y - mu), axis=-1, keepdims=True)
    return (y - mu) * lax.rsqrt(var + 1e-5) * g + b


def _w2(shape):
    return pl.BlockSpec(shape, lambda i: (0, 0))


def _w3(shape):
    return pl.BlockSpec(shape, lambda i: (0, 0, 0))


def _enc_node_body(xo, xc, w1, b1, w2, b2, w3, b3, g, bln, wp,
                   v_ref, p0, p1, p2, p3):
    i = pl.program_id(0)
    x = jnp.where(i < 2, xo[...], xc[...])
    _enc_proj_body_from(x, w1, b1, w2, b2, w3, b3, g, bln, wp,
                        v_ref, p0, p1, p2, p3)


def _enc_proj_body_from(x, w1, b1, w2, b2, w3, b3, g, bln, wp,
                        v_ref, p0, p1, p2, p3):
    h = jnp.maximum(_dot(x, w1[...]) + b1[...], 0.0)
    h = jnp.maximum(_dot(h, w2[...]) + b2[...], 0.0)
    v = _ln(_dot(h, w3[...]) + b3[...], g[...], bln[...])
    v_ref[...] = v
    p0[...] = _dot(v, wp[0])
    p1[...] = _dot(v, wp[1])
    p2[...] = _dot(v, wp[2])
    p3[...] = _dot(v, wp[3])


def _enc_node(xo, xc, w, wp):
    w1, w2, w3, b1, b2, b3, g, bln = w
    bn = 1000
    outs = [jax.ShapeDtypeStruct((NV, L), F32) for _ in range(5)]
    return pl.pallas_call(
        _enc_node_body,
        grid=(12,),
        in_specs=[pl.BlockSpec((bn, 12), lambda i: (jnp.minimum(i, 1), 0)),
                  pl.BlockSpec((bn, 12), lambda i: (jnp.maximum(i - 2, 0), 0)),
                  _w2((12, L)), _w2((1, L)), _w2((L, L)), _w2((1, L)),
                  _w2((L, L)), _w2((1, L)), _w2((1, L)), _w2((1, L)),
                  _w3((4, L, L))],
        out_specs=[pl.BlockSpec((bn, L), lambda i: (i, 0))] * 5,
        out_shape=outs,
    )(xo, xc, w1, b1, w2, b2, w3, b3, g, bln, wp)


def _enc_edge_body(x_ref, w1, b1, w2, b2, w3, b3, g, bln, e_ref):
    h = jnp.maximum(_bdot(x_ref[...], w1[...]) + b1[...], 0.0)
    h = jnp.maximum(_bdot(h, w2[...]) + b2[...], 0.0)
    e_ref[...] = _ln(_bdot(h, w3[...]) + b3[...], g[...],
                     bln[...]).astype(jnp.bfloat16)


def _enc_edge(x, w, n_out):
    w1, w2, w3, b1, b2, b3, g, bln = w
    f = x.shape[1]
    grid = (x.shape[0] + BE - 1) // BE
    return pl.pallas_call(
        _enc_edge_body,
        grid=(grid,),
        in_specs=[pl.BlockSpec((BE, f), lambda i: (i, 0)),
                  _w2((f, L)), _w2((1, L)), _w2((L, L)), _w2((1, L)),
                  _w2((L, L)), _w2((1, L)), _w2((1, L)), _w2((1, L))],
        out_specs=pl.BlockSpec((BE, L), lambda i: (i, 0)),
        out_shape=jax.ShapeDtypeStruct((n_out, L), jnp.bfloat16),
    )(x, w1, b1, w2, b2, w3, b3, g, bln)


def _enc_world_body(xd, xi, w1, b1, w2, b2, w3, b3, g, bln, e_ref):
    i = pl.program_id(0)
    x = jnp.where(i < 10, xd[...], xi[...])
    h = jnp.maximum(_bdot(x, w1[...]) + b1[...], 0.0)
    h = jnp.maximum(_bdot(h, w2[...]) + b2[...], 0.0)
    e_ref[...] = _ln(_bdot(h, w3[...]) + b3[...], g[...],
                     bln[...]).astype(jnp.bfloat16)


def _enc_world(xd, xi, w):
    w1, w2, w3, b1, b2, b3, g, bln = w
    be = 2000
    return pl.pallas_call(
        _enc_world_body,
        grid=(20,),
        in_specs=[pl.BlockSpec((be, 4), lambda i: (jnp.minimum(i, 9), 0)),
                  pl.BlockSpec((be, 4), lambda i: (jnp.maximum(i - 10, 0), 0)),
                  _w2((4, L)), _w2((1, L)), _w2((L, L)), _w2((1, L)),
                  _w2((L, L)), _w2((1, L)), _w2((1, L)), _w2((1, L))],
        out_specs=pl.BlockSpec((be, L), lambda i: (i, 0)),
        out_shape=jax.ShapeDtypeStruct((EW_P, L), jnp.bfloat16),
    )(xd, xi, w1, b1, w2, b2, w3, b3, g, bln)


def _edge_upd_body(ga, gb, e_ref, w1c, b1, w2, b2, w3, b3, g, bln,
                   u_ref, en_ref):
    e = e_ref[...]
    x = jnp.maximum(ga[...] + gb[...] + jnp.dot(
        e, w1c[...], preferred_element_type=F32) + b1[...], 0.0)
    h = jnp.maximum(_bdot(x, w2[...]) + b2[...], 0.0)
    u = _ln(_bdot(h, w3[...]) + b3[...], g[...], bln[...])
    u_ref[...] = u
    en_ref[...] = (e.astype(F32) + u).astype(jnp.bfloat16)


def _edge_upd(ga, gb, e, w):
    w1a, w1b, w1c, w2, w3, b1, b2, b3, g, bln = w
    bf = jnp.bfloat16
    w1c, w2, w3 = w1c.astype(bf), w2.astype(bf), w3.astype(bf)
    n = e.shape[0]
    outs = [jax.ShapeDtypeStruct((n, L), F32),
            jax.ShapeDtypeStruct((n, L), bf)]
    blk = pl.BlockSpec((BE, L), lambda i: (i, 0))
    return pl.pallas_call(
        _edge_upd_body,
        grid=(n // BE,),
        in_specs=[blk, blk, blk,
                  _w2((L, L)), _w2((1, L)), _w2((L, L)), _w2((1, L)),
                  _w2((L, L)), _w2((1, L)), _w2((1, L)), _w2((1, L))],
        out_specs=[blk, blk],
        out_shape=outs,
    )(ga, gb, e, w1c, b1, w2, b2, w3, b3, g, bln)


def _node_core(a1, a2, v_ref, w1a, w1b, w1c, b1, w2, b2, w3, b3, g, bln):
    v = v_ref[...]
    A1 = a1[0] + a1[1]
    A2 = a2[0] + a2[1]
    x = jnp.maximum(_dot(A1, w1a[...]) + _dot(A2, w1b[...])
                    + _dot(v, w1c[...]) + b1[...], 0.0)
    h = jnp.maximum(_dot(x, w2[...]) + b2[...], 0.0)
    return v + _ln(_dot(h, w3[...]) + b3[...], g[...], bln[...])


def _node_proj_body(a1, a2, v_ref, w1a, w1b, w1c, b1, w2, b2, w3, b3, g, bln,
                    wp, v_out, p0, p1, p2, p3):
    vn = _node_core(a1, a2, v_ref, w1a, w1b, w1c, b1, w2, b2, w3, b3, g, bln)
    v_out[...] = vn
    p0[...] = _dot(vn, wp[0])
    p1[...] = _dot(vn, wp[1])
    p2[...] = _dot(vn, wp[2])
    p3[...] = _dot(vn, wp[3])


def _node_dec_body(a1, a2, v_ref, w1a, w1b, w1c, b1, w2, b2, w3, b3, g, bln,
                   dw1, db1, dw2, db2, dw3, db3, dec_ref):
    vn = _node_core(a1, a2, v_ref, w1a, w1b, w1c, b1, w2, b2, w3, b3, g, bln)
    d = jnp.maximum(_dot(vn, dw1[...]) + db1[...], 0.0)
    d = jnp.maximum(_dot(d, dw2[...]) + db2[...], 0.0)
    dec_ref[...] = _dot(d, dw3[...]) + db3[...]


_AGG_SPEC = pl.BlockSpec((2, BN, L), lambda i: (0, i, 0))
_NODE_W_SPECS = [_w2((L, L)), _w2((L, L)), _w2((L, L)), _w2((1, L)),
                 _w2((L, L)), _w2((1, L)), _w2((L, L)), _w2((1, L)),
                 _w2((1, L)), _w2((1, L))]


def _node_proj(agg1, agg2, v, w, wp):
    w1a, w1b, w1c, w2, w3, b1, b2, b3, g, bln = w
    blk = pl.BlockSpec((BN, L), lambda i: (i, 0))
    outs = [jax.ShapeDtypeStruct((NV, L), F32)] * 5
    return pl.pallas_call(
        _node_proj_body,
        grid=(NV // BN,),
        in_specs=[_AGG_SPEC, _AGG_SPEC, blk] + _NODE_W_SPECS + [_w3((4, L, L))],
        out_specs=[blk] * 5,
        out_shape=outs,
    )(agg1, agg2, v, w1a, w1b, w1c, b1, w2, b2, w3, b3, g, bln, wp)


def _node_dec(agg1, agg2, v, w, dw):
    w1a, w1b, w1c, w2, w3, b1, b2, b3, g, bln = w
    dw1, dw2, dw3, db1, db2, db3 = dw
    blk = pl.BlockSpec((BN, L), lambda i: (i, 0))
    return pl.pallas_call(
        _node_dec_body,
        grid=(NV // BN,),
        in_specs=[_AGG_SPEC, _AGG_SPEC, blk] + _NODE_W_SPECS
        + [_w2((L, L)), _w2((1, L)), _w2((L, L)), _w2((1, L)),
           _w2((L, 8)), _w2((1, 8))],
        out_specs=pl.BlockSpec((BN, 8), lambda i: (i, 0)),
        out_shape=jax.ShapeDtypeStruct((NV, 8), F32),
    )(agg1, agg2, v, w1a, w1b, w1c, b1, w2, b2, w3, b3, g, bln,
      dw1, db1, dw2, db2, dw3, db3)


GCH = 64


def _pipe_gather(tbl, ivm, out, nch, base, bufs, gsems, ssems):
    def idx(c):
        return ivm.at[c // 2, pl.ds((c % 2) * 64, GCH)]

    for k in range(2):
        pltpu.async_copy(tbl.at[idx(k)], bufs[k], gsems[k])

    def body(i, carry):
        c0 = i * 2
        for k in range(2):
            c = c0 + k
            pltpu.make_async_copy(tbl.at[idx(c)], bufs[k], gsems[k]).wait()
            dst = out.at[pl.ds(base + c * GCH, GCH)]
            pltpu.async_copy(bufs[k], dst, ssems[k])

            @pl.when(c + 2 < nch)
            def _():
                pltpu.make_async_copy(bufs[k], dst, ssems[k]).wait()
                pltpu.async_copy(tbl.at[idx(c + 2)], bufs[k], gsems[k])
        return carry
    lax.fori_loop(0, nch // 2, body, 0)
    for k in range(2):
        c = nch - 2 + k
        pltpu.make_async_copy(
            bufs[k], out.at[pl.ds(base + c * GCH, GCH)], ssems[k]).wait()


def _stage(src, dst, s):
    @pl.when(s < 15)
    def _():
        pltpu.sync_copy(src.at[pl.ds(s * 752, 752)],
                        dst.at[pl.ds(s * 752, 752)])

    @pl.when(s == 15)
    def _():
        pltpu.sync_copy(src.at[pl.ds(11280, 720)],
                        dst.at[pl.ds(11280, 720)])


def _gather_body(p0, p1, p2, p3, imA, imB, iwA, iwB,
                 gmA, gmB, gwA, gwB,
                 vimA, vimB, viwA, viwB,
                 b0, b1, tbl_sh, g0, g1, s0, s1):
    c = lax.axis_index("c")
    s = lax.axis_index("s")
    wid = c * 16 + s
    pltpu.sync_copy(imA.at[wid], vimA)
    pltpu.sync_copy(imB.at[wid], vimB)
    pltpu.sync_copy(iwA.at[wid], viwA)
    pltpu.sync_copy(iwB.at[wid], viwB)
    bufs = (b0, b1)
    gsems = (g0, g1)
    ssems = (s0, s1)
    mbase = wid * (M_CH * 128)
    wbase = wid * (W_CH * 128)
    mch = (M_CH * 128) // GCH
    wch = (W_CH * 128) // GCH
    for tbl, ivm, out, nch, base in ((p0, vimA, gmA, mch, mbase),
                                     (p1, vimB, gmB, mch, mbase),
                                     (p2, viwA, gwA, wch, wbase),
                                     (p3, viwB, gwB, wch, wbase)):
        _stage(tbl, tbl_sh, s)
        plsc.subcore_barrier()
        _pipe_gather(tbl_sh, ivm, out, nch, base, bufs, gsems, ssems)
        plsc.subcore_barrier()


@functools.cache
def _sc_gather_kernel():
    return pl.kernel(
        _gather_body,
        out_type=[jax.ShapeDtypeStruct((EM_P, L), F32),
                  jax.ShapeDtypeStruct((EM_P, L), F32),
                  jax.ShapeDtypeStruct((EW_P, L), F32),
                  jax.ShapeDtypeStruct((EW_P, L), F32)],
        mesh=_sc_mesh(),
        scratch_types=[pltpu.VMEM((M_CH, 128), jnp.int32),
                       pltpu.VMEM((M_CH, 128), jnp.int32),
                       pltpu.VMEM((W_CH, 128), jnp.int32),
                       pltpu.VMEM((W_CH, 128), jnp.int32)]
        + [pltpu.VMEM((GCH, L), F32)] * 2
        + [pltpu.VMEM_SHARED((NV, L), F32)]
        + [pltpu.SemaphoreType.DMA] * 4,
    )


def _sc_gather(*args):
    return _sc_gather_kernel()(*args)


def _make_scatter(e_p, nch):
    npt = e_p // NW
    sch = 64

    def body(u_ref, idx_ref, out_ref, ivm, d0, d1, acc, rs0, rs1, ws0, ws1):
        c = lax.axis_index("c")
        s = lax.axis_index("s")
        wid = c * 16 + s
        bufs = (d0, d1)

        def zb(i, carry):
            r = i // 8
            col = (i % 8) * 16
            d0[r, pl.ds(col, 16)] = jnp.zeros((16,), F32)
            return carry
        lax.fori_loop(0, 512, zb, 0)

        def za(r, carry):
            pltpu.sync_copy(d0, acc.at[pl.ds(s * 800 + r * sch, sch)])
            return carry
        lax.fori_loop(0, 12, za, 0)
        pltpu.sync_copy(d0.at[pl.ds(0, 32)],
                        acc.at[pl.ds(s * 800 + 768, 32)])
        plsc.subcore_barrier()

        pltpu.sync_copy(idx_ref.at[wid], ivm)

        def src(ci):
            return u_ref.at[pl.ds(wid * npt + ci * sch, sch)]

        rsems = (rs0, rs1)
        wsems = (ws0, ws1)
        for k in range(2):
            pltpu.async_copy(src(k), bufs[k], rsems[k])

        def sc(i, carry):
            c0 = i * 2
            for k in range(2):
                ci = c0 + k
                pltpu.make_async_copy(src(ci), bufs[k], rsems[k]).wait()
                pltpu.async_copy(bufs[k], acc.at[ivm.at[ci]], wsems[k],
                                 add=True)

                @pl.when(ci + 2 < nch)
                def _():
                    pltpu.make_async_copy(bufs[k], acc.at[ivm.at[ci]],
                                          wsems[k]).wait()
                    pltpu.async_copy(src(ci + 2), bufs[k], rsems[k])
            return carry
        lax.fori_loop(0, nch // 2, sc, 0)
        for k in range(2):
            ci = nch - 2 + k
            pltpu.make_async_copy(bufs[k], acc.at[ivm.at[ci]],
                                  wsems[k]).wait()
        plsc.subcore_barrier()

        pltpu.sync_copy(acc.at[pl.ds(s * 800, 800)],
                        out_ref.at[pl.ds(c * ACC_R + s * 800, 800)])

    @functools.cache
    def build():
        return pl.kernel(
            body,
            out_type=jax.ShapeDtypeStruct((2 * ACC_R, L), F32),
            mesh=_sc_mesh(),
            scratch_types=[pltpu.VMEM((nch, sch), jnp.int32),
                           pltpu.VMEM((sch, L), F32),
                           pltpu.VMEM((sch, L), F32),
                           pltpu.VMEM_SHARED((ACC_R, L), F32)]
            + [pltpu.SemaphoreType.DMA] * 4,
        )

    def call(u, idx3):
        return build()(u, idx3)

    return call


_sc_scatter_mesh = _make_scatter(EM_P, 80)
_sc_scatter_world = _make_scatter(EW_P, 20)


def _mw(mp):
    W1, W2, W3 = mp['Ws']
    b1, b2, b3 = [b.reshape(1, L) for b in mp['bs']]
    return W1, W2, W3, b1, b2, b3, mp['g'].reshape(1, L), mp['b2'].reshape(1, L)


def _enc_w_bf(mp):
    W1, W2, W3, b1, b2, b3, g, bln = _mw(mp)
    bf = jnp.bfloat16
    return W1.astype(bf), W2.astype(bf), W3.astype(bf), b1, b2, b3, g, bln


def _blk_w(mp):
    W1, W2, W3, b1, b2, b3, g, bln = _mw(mp)
    return W1[:L], W1[L:2 * L], W1[2 * L:], W2, W3, b1, b2, b3, g, bln


def _proj_stack(blk):
    wm = blk['mesh_edge']['Ws'][0]
    ww = blk['world_edge']['Ws'][0]
    return jnp.stack([wm[:L], wm[L:2 * L], ww[:L], ww[L:2 * L]])


def kernel(cloth_features, obstacle_features, mesh_edge_features,
           world_direct_features, world_inverse_features, params,
           mesh_edge_index, world_direct_edge_index, world_inverse_edge_index):
    p = params


    mdst = mesh_edge_index[1] + NO
    msrc = mesh_edge_index[0] + NO
    im_a = jnp.pad(mdst, (0, EM_P - 160000)).reshape(NW, M_CH, 128)
    im_b = jnp.pad(msrc, (0, EM_P - 160000)).reshape(NW, M_CH, 128)
    im_s = jnp.pad(mdst, (0, EM_P - 160000),
                   constant_values=DUMMY).reshape(NW, 80, 64)
    wdst = jnp.concatenate([world_direct_edge_index[1],
                            world_inverse_edge_index[1] + NO])
    wsrc = jnp.concatenate([world_direct_edge_index[0] + NO,
                            world_inverse_edge_index[0]])
    iw_a = jnp.pad(wdst, (0, EW_P - 40000)).reshape(NW, W_CH, 128)
    iw_b = jnp.pad(wsrc, (0, EW_P - 40000)).reshape(NW, W_CH, 128)
    iw_s = jnp.pad(wdst, (0, EW_P - 40000),
                   constant_values=DUMMY).reshape(NW, 20, 64)

    enc_node_w = _mw(p['node_encoder'])
    enc_mesh_w = _enc_w_bf(p['mesh_encoder'])
    enc_world_w = _enc_w_bf(p['world_encoder'])
    blk_w = [{'mesh': _blk_w(b['mesh_edge']),
              'world': _blk_w(b['world_edge']),
              'node': _blk_w(b['node'])} for b in p['blocks']]
    dW1, dW2, dW3 = p['decoder']['Ws']
    dec_w = (dW1, dW2, jnp.pad(dW3, ((0, 0), (0, 8 - dW3.shape[1]))),
             p['decoder']['bs'][0].reshape(1, L),
             p['decoder']['bs'][1].reshape(1, L),
             jnp.pad(p['decoder']['bs'][2].reshape(1, 3), ((0, 0), (0, 5))))

    v, p0, p1, p2, p3 = _enc_node(obstacle_features, cloth_features,
                                  enc_node_w, _proj_stack(p['blocks'][0]))
    em = _enc_edge(mesh_edge_features, enc_mesh_w, EM_P)
    ew = _enc_world(world_direct_features, world_inverse_features,
                    enc_world_w)

    for s in range(len(p['blocks'])):
        gmA, gmB, gwA, gwB = _sc_gather(p0, p1, p2, p3, im_a, im_b, iw_a, iw_b)
        em_upd, em = _edge_upd(gmA, gmB, em, blk_w[s]['mesh'])
        ew_upd, ew = _edge_upd(gwA, gwB, ew, blk_w[s]['world'])
        agg1 = _sc_scatter_mesh(em_upd, im_s).reshape(2, ACC_R, L)
        agg2 = _sc_scatter_world(ew_upd, iw_s).reshape(2, ACC_R, L)
        if s + 1 < len(p['blocks']):
            v, p0, p1, p2, p3 = _node_proj(agg1, agg2, v, blk_w[s]['node'],
                                           _proj_stack(p['blocks'][s + 1]))
        else:
            dec = _node_dec(agg1, agg2, v, blk_w[s]['node'], dec_w)

    return dec[NO:NV, :3]
